# baseline (speedup 1.0000x reference)
import functools

import jax
import jax.numpy as jnp
from jax import lax
from jax.experimental import pallas as pl
from jax.experimental.pallas import tpu as pltpu
from jax.experimental.pallas import tpu_sc as plsc

BATCH = 4
SEQ = 8192
DIM = 32
ORIG_PE = 4096
N = BATCH * SEQ

NUM_CORES = 2
NUM_SUBCORES = 16
NUM_WORKERS = NUM_CORES * NUM_SUBCORES
LANES = 16

CHUNK = N // NUM_WORKERS
IDX_ROW = 128
ROWS_PER_WORKER = CHUNK // IDX_ROW

_mesh = plsc.VectorSubcoreMesh(core_axis_name="c", subcore_axis_name="s")


@functools.partial(
    pl.kernel,
    mesh=_mesh,
    compiler_params=pltpu.CompilerParams(needs_layout_passes=False,
                                         use_tc_tiling_on_sc=False),
    out_type=[
        jax.ShapeDtypeStruct((N, DIM), jnp.float32),
        jax.ShapeDtypeStruct((N, DIM), jnp.float32),
    ],
    scratch_types=[
        pltpu.VMEM((ROWS_PER_WORKER, IDX_ROW), jnp.int32),
        pltpu.VMEM((ROWS_PER_WORKER, IDX_ROW), jnp.int32),
        pltpu.VMEM((CHUNK, DIM), jnp.float32),
        pltpu.VMEM((CHUNK, DIM), jnp.float32),
        pltpu.VMEM((LANES,), jnp.int32),
        pltpu.VMEM((NUM_SUBCORES, LANES), jnp.int32),
        pltpu.VMEM_SHARED((NUM_SUBCORES, LANES), jnp.int32),
        pltpu.SemaphoreType.DMA,
    ],
)
def _rope_gather(pids_hbm, cos_hbm, sin_hbm, cos_out, sin_out,
                 own_idx, oth_idx, cos_rows, sin_rows,
                 maxv, allmax, shared_max, sem):
    c = lax.axis_index("c")
    s = lax.axis_index("s")
    wid = s * NUM_CORES + c
    owid = s * NUM_CORES + (1 - c)

    pltpu.sync_copy(pids_hbm.at[pl.ds(wid * ROWS_PER_WORKER, ROWS_PER_WORKER)], own_idx)
    pltpu.sync_copy(pids_hbm.at[pl.ds(owid * ROWS_PER_WORKER, ROWS_PER_WORKER)], oth_idx)

    m = own_idx[0, pl.ds(0, LANES)]
    for buf in (own_idx, oth_idx):
        for r in range(ROWS_PER_WORKER):
            for k in range(IDX_ROW // LANES):
                m = jnp.maximum(m, buf[r, pl.ds(k * LANES, LANES)])

    maxv[...] = m
    pltpu.sync_copy(maxv, shared_max.at[s])
    plsc.subcore_barrier()
    pltpu.sync_copy(shared_max, allmax)
    g = allmax[0, pl.ds(0, LANES)]
    for r in range(1, NUM_SUBCORES):
        g = jnp.maximum(g, allmax[r, pl.ds(0, LANES)])
    lane = lax.iota(jnp.int32, LANES)
    for sh in (8, 4, 2, 1):
        maxv[...] = g
        g = jnp.maximum(g, plsc.load_gather(maxv, [(lane + sh) & (LANES - 1)]))
    off = jnp.where(g >= ORIG_PE, jnp.int32(ORIG_PE), jnp.int32(0))

    for r in range(ROWS_PER_WORKER):
        for k in range(IDX_ROW // LANES):
            sl = pl.ds(k * LANES, LANES)
            own_idx[r, sl] = own_idx[r, sl] + off

    copies = []
    for r in range(ROWS_PER_WORKER):
        dst = pl.ds(r * IDX_ROW, IDX_ROW)
        copies.append(pltpu.async_copy(cos_hbm.at[own_idx.at[r]], cos_rows.at[dst], sem))
        copies.append(pltpu.async_copy(sin_hbm.at[own_idx.at[r]], sin_rows.at[dst], sem))
    for cp in copies:
        cp.wait()

    base = wid * CHUNK
    pltpu.sync_copy(cos_rows, cos_out.at[pl.ds(base, CHUNK)])
    pltpu.sync_copy(sin_rows, sin_out.at[pl.ds(base, CHUNK)])


def kernel(position_ids, cos_cache, sin_cache):
    pids = position_ids.reshape(N // IDX_ROW, IDX_ROW)
    cos_flat, sin_flat = _rope_gather(pids, cos_cache, sin_cache)
    return (cos_flat.reshape(BATCH, SEQ, DIM),
            sin_flat.reshape(BATCH, SEQ, DIM))
